# Initial kernel scaffold; baseline (speedup 1.0000x reference)
import jax
import jax.numpy as jnp
from jax import lax
from jax.experimental import pallas as pl
from jax.experimental.pallas import tpu as pltpu
from jax.experimental.pallas import tpu_sc as plsc

N_USERS = 25000
N_ITEMS = 25000
N_NODES = 50000
EMB = 64
N_LAYERS = 3

NC = 2
NS = 16
CH = 512
SUB = 128
EP = 802816
EPT = EP // NS
NCHUNK = EPT // CH

HALF0 = 25088
ROWS_PT = HALF0 // NS
NPAD = 2 * HALF0
DUMP = 64
ACC_ROWS = HALF0 + DUMP
ZPT = ACC_ROWS // NS


def _layer_body(x_hbm, src2_hbm, dst_hbm, w_hbm, out_hbm,
                acc, sidx, dstb, wb, dloc, rows, sem):
    c = lax.axis_index("c")
    s = lax.axis_index("s")
    lo = c * HALF0
    hi = HALF0 + c * (N_NODES - HALF0)

    def zrow(i, carry):
        for k in range(4):
            rows[i, pl.ds(k * 16, 16)] = jnp.zeros((16,), jnp.float32)
        return carry
    lax.fori_loop(0, CH, zrow, 0)
    z0 = s * ZPT
    pltpu.sync_copy(rows.at[pl.ds(0, 512)], acc.at[pl.ds(z0, 512)])
    pltpu.sync_copy(rows.at[pl.ds(0, 512)], acc.at[pl.ds(z0 + 512, 512)])
    pltpu.sync_copy(rows.at[pl.ds(0, 512)], acc.at[pl.ds(z0 + 1024, 512)])
    pltpu.sync_copy(rows.at[pl.ds(0, ZPT - 1536)], acc.at[pl.ds(z0 + 1536, ZPT - 1536)])
    plsc.subcore_barrier()

    def chunk(i, carry):
        base = s * EPT + i * CH
        row0 = s * (EPT // SUB) + i * (CH // SUB)
        pltpu.sync_copy(src2_hbm.at[pl.ds(row0, CH // SUB)], sidx)
        pltpu.sync_copy(dst_hbm.at[pl.ds(base, CH)], dstb)
        pltpu.sync_copy(w_hbm.at[pl.ds(base, CH)], wb)
        descs = [pltpu.async_copy(x_hbm.at[sidx.at[t]],
                                  rows.at[pl.ds(t * SUB, SUB)], sem)
                 for t in range(CH // SUB)]
        for d in descs:
            d.wait()
        for t in range(CH // SUB):
            for g in range(SUB // 16):
                o = t * SUB + g * 16
                d16 = dstb[pl.ds(o, 16)]
                inr = (d16 >= lo) & (d16 < hi)
                dl = jnp.where(inr, d16 - lo, HALF0 + (d16 & (DUMP - 1)))
                dloc[t, pl.ds(g * 16, 16)] = dl
        def blk(b, carry2):
            jb = b * 8
            for e in range(8):
                j = jb + e
                ws = plsc.load_gather(wb, [jnp.full((16,), j, jnp.int32)])
                for k in range(4):
                    rows[j, pl.ds(k * 16, 16)] = rows[j, pl.ds(k * 16, 16)] * ws
            return carry2
        lax.fori_loop(0, CH // 8, blk, 0)
        for t in range(CH // SUB):
            pltpu.sync_copy(rows.at[pl.ds(t * SUB, SUB)],
                            acc.at[dloc.at[t]], add=True)
        return carry
    lax.fori_loop(0, NCHUNK, chunk, 0)
    plsc.subcore_barrier()

    w0 = s * ROWS_PT
    g0 = c * HALF0 + w0
    for q in range(3):
        pltpu.sync_copy(acc.at[pl.ds(w0 + q * 512, 512)],
                        out_hbm.at[pl.ds(g0 + q * 512, 512)])
    pltpu.sync_copy(acc.at[pl.ds(w0 + 1536, ROWS_PT - 1536)],
                    out_hbm.at[pl.ds(g0 + 1536, ROWS_PT - 1536)])


_sc_layer = pl.kernel(
    _layer_body,
    out_type=jax.ShapeDtypeStruct((NPAD, EMB), jnp.float32),
    mesh=plsc.VectorSubcoreMesh(core_axis_name="c", subcore_axis_name="s"),
    scratch_types=[
        pltpu.VMEM_SHARED((ACC_ROWS, EMB), jnp.float32),
        pltpu.VMEM((CH // SUB, SUB), jnp.int32),
        pltpu.VMEM((CH,), jnp.int32),
        pltpu.VMEM((CH,), jnp.float32),
        pltpu.VMEM((CH // SUB, SUB), jnp.int32),
        pltpu.VMEM((CH, EMB), jnp.float32),
        pltpu.SemaphoreType.DMA,
    ],
)


def _mean_body(a, b, c, d, o):
    o[...] = (a[...] + b[...] + c[...] + d[...]) * 0.25


_mean = pl.pallas_call(
    _mean_body,
    grid=(250,),
    in_specs=[pl.BlockSpec((200, EMB), lambda i: (i, 0))] * 4,
    out_specs=pl.BlockSpec((200, EMB), lambda i: (i, 0)),
    out_shape=jax.ShapeDtypeStruct((N_NODES, EMB), jnp.float32),
)


def kernel(user_weight, item_weight, edge_index, edge_weight):
    x0 = jnp.concatenate(
        [user_weight, item_weight,
         jnp.zeros((NPAD - N_NODES, EMB), jnp.float32)], axis=0)
    src = edge_index[1]
    dst = edge_index[0]
    pad = EP - src.shape[0]
    src_p = jnp.concatenate([src, jnp.zeros((pad,), jnp.int32)])
    dst_p = jnp.concatenate([dst, jnp.full((pad,), N_NODES, jnp.int32)])
    w_p = jnp.concatenate([edge_weight, jnp.zeros((pad,), jnp.float32)])
    src2 = src_p.reshape(EP // SUB, SUB)

    xs = [x0]
    cur = x0
    for _ in range(N_LAYERS):
        cur = _sc_layer(cur, src2, dst_p, w_p)
        xs.append(cur)
    return _mean(*xs)


# speedup vs baseline: 2.1375x; 2.1375x over previous
import jax
import jax.numpy as jnp
from jax import lax
from jax.experimental import pallas as pl
from jax.experimental.pallas import tpu as pltpu
from jax.experimental.pallas import tpu_sc as plsc

N_USERS = 25000
N_ITEMS = 25000
N_NODES = 50000
EMB = 64
N_LAYERS = 3

NC = 2
NS = 16
CH = 256
SUB = 128
EP = 802816
EPT = EP // NS
NCHUNK = EPT // CH

HALF0 = 25088
ROWS_PT = HALF0 // NS
NPAD = 2 * HALF0
DUMP = 64
ACC_ROWS = HALF0 + DUMP
ZPT = ACC_ROWS // NS


_GATHER_DNUMS = lax.GatherDimensionNumbers(
    offset_dims=(), collapsed_slice_dims=(0,), start_index_map=(0,))


def _lane_splat(v16, e):
    idx = jnp.full((16, 1), e, jnp.int32)
    return lax.gather(v16, idx, _GATHER_DNUMS, (1,),
                      mode=lax.GatherScatterMode.PROMISE_IN_BOUNDS)


def _layer_body(x_hbm, src2_hbm, dst_hbm, w_hbm, out_hbm,
                acc, sidx, dstb, wb, dloc, rows, sem):
    c = lax.axis_index("c")
    s = lax.axis_index("s")
    lo = c * HALF0
    hi = HALF0 + c * (N_NODES - HALF0)

    def zrow(i, carry):
        for k in range(4):
            rows[i, pl.ds(k * 16, 16)] = jnp.zeros((16,), jnp.float32)
        return carry
    lax.fori_loop(0, CH, zrow, 0)
    z0 = s * ZPT
    nfull, rem = divmod(ZPT, CH)
    for q in range(nfull):
        pltpu.sync_copy(rows.at[pl.ds(0, CH)], acc.at[pl.ds(z0 + q * CH, CH)])
    if rem:
        pltpu.sync_copy(rows.at[pl.ds(0, rem)], acc.at[pl.ds(z0 + nfull * CH, rem)])
    plsc.subcore_barrier()

    def chunk(i, carry):
        base = s * EPT + i * CH
        row0 = s * (EPT // SUB) + i * (CH // SUB)
        pltpu.sync_copy(src2_hbm.at[pl.ds(row0, CH // SUB)], sidx)
        pltpu.sync_copy(dst_hbm.at[pl.ds(base, CH)], dstb)
        pltpu.sync_copy(w_hbm.at[pl.ds(base, CH)], wb)
        descs = [pltpu.async_copy(x_hbm.at[sidx.at[t]],
                                  rows.at[pl.ds(t * SUB, SUB)], sem)
                 for t in range(CH // SUB)]
        for d in descs:
            d.wait()
        for t in range(CH // SUB):
            for g in range(SUB // 16):
                o = t * SUB + g * 16
                d16 = dstb[pl.ds(o, 16)]
                inr = (d16 >= lo) & (d16 < hi)
                dl = jnp.where(inr, d16 - lo, HALF0 + (d16 & (DUMP - 1)))
                dloc[t, pl.ds(g * 16, 16)] = dl
        def blk(b, carry2):
            jb = b * 16
            w16 = wb[pl.ds(jb, 16)]
            for e in range(16):
                j = jb + e
                ws = _lane_splat(w16, e)
                for k in range(4):
                    rows[j, pl.ds(k * 16, 16)] = rows[j, pl.ds(k * 16, 16)] * ws
            return carry2
        lax.fori_loop(0, CH // 16, blk, 0)
        for t in range(CH // SUB):
            pltpu.sync_copy(rows.at[pl.ds(t * SUB, SUB)],
                            acc.at[dloc.at[t]], add=True)
        return carry
    lax.fori_loop(0, NCHUNK, chunk, 0)
    plsc.subcore_barrier()

    w0 = s * ROWS_PT
    g0 = c * HALF0 + w0
    nfullw, remw = divmod(ROWS_PT, CH)
    for q in range(nfullw):
        pltpu.sync_copy(acc.at[pl.ds(w0 + q * CH, CH)],
                        out_hbm.at[pl.ds(g0 + q * CH, CH)])
    if remw:
        pltpu.sync_copy(acc.at[pl.ds(w0 + nfullw * CH, remw)],
                        out_hbm.at[pl.ds(g0 + nfullw * CH, remw)])


import functools


@functools.cache
def _sc_layer_fn():
  return pl.kernel(
    _layer_body,
    out_type=jax.ShapeDtypeStruct((NPAD, EMB), jnp.float32),
    mesh=plsc.VectorSubcoreMesh(core_axis_name="c", subcore_axis_name="s",
                                num_cores=NC, num_subcores=NS),
    compiler_params=pltpu.CompilerParams(use_tc_tiling_on_sc=False),
    scratch_types=[
        pltpu.VMEM_SHARED((ACC_ROWS, EMB), jnp.float32),
        pltpu.VMEM((CH // SUB, SUB), jnp.int32),
        pltpu.VMEM((CH,), jnp.int32),
        pltpu.VMEM((CH,), jnp.float32),
        pltpu.VMEM((CH // SUB, SUB), jnp.int32),
        pltpu.VMEM((CH, EMB), jnp.float32),
        pltpu.SemaphoreType.DMA,
    ],
  )


def _mean_body(a, b, c, d, o):
    o[...] = (a[...] + b[...] + c[...] + d[...]) * 0.25


_mean = pl.pallas_call(
    _mean_body,
    grid=(250,),
    in_specs=[pl.BlockSpec((200, EMB), lambda i: (i, 0))] * 4,
    out_specs=pl.BlockSpec((200, EMB), lambda i: (i, 0)),
    out_shape=jax.ShapeDtypeStruct((N_NODES, EMB), jnp.float32),
)


def kernel(user_weight, item_weight, edge_index, edge_weight):
    x0 = jnp.concatenate(
        [user_weight, item_weight,
         jnp.zeros((NPAD - N_NODES, EMB), jnp.float32)], axis=0)
    src = edge_index[1]
    dst = edge_index[0]
    pad = EP - src.shape[0]
    src_p = jnp.concatenate([src, jnp.zeros((pad,), jnp.int32)])
    dst_p = jnp.concatenate([dst, jnp.full((pad,), N_NODES, jnp.int32)])
    w_p = jnp.concatenate([edge_weight, jnp.zeros((pad,), jnp.float32)])
    src2 = src_p.reshape(EP // SUB, SUB)

    sc_layer = _sc_layer_fn()
    xs = [x0]
    cur = x0
    for _ in range(N_LAYERS):
        cur = sc_layer(cur, src2, dst_p, w_p)
        xs.append(cur)
    return _mean(*xs)


# speedup vs baseline: 2.3884x; 1.1173x over previous
import functools

import jax
import jax.numpy as jnp
from jax import lax
from jax.experimental import pallas as pl
from jax.experimental.pallas import tpu as pltpu
from jax.experimental.pallas import tpu_sc as plsc

N_USERS = 25000
N_ITEMS = 25000
N_NODES = 50000
EMB = 64
N_LAYERS = 3

NC = 2
NS = 16
CH = 128
EP = 802816
EPT = EP // NS
NCH = EPT // CH

HALF0 = 25088
ROWS_PT = HALF0 // NS
NPAD = 2 * HALF0
DUMP = 64
ACC_ROWS = HALF0 + DUMP
ZPT = ACC_ROWS // NS

_GATHER_DNUMS = lax.GatherDimensionNumbers(
    offset_dims=(), collapsed_slice_dims=(0,), start_index_map=(0,))


def _lane_splat(v16, e):
    idx = jnp.full((16, 1), e, jnp.int32)
    return lax.gather(v16, idx, _GATHER_DNUMS, (1,),
                      mode=lax.GatherScatterMode.PROMISE_IN_BOUNDS)


def _layer_body(x_hbm, e2_hbm, w2_hbm, out_hbm,
                acc, ebuf, wbuf, dloc, rows2, sg0, sg1, ss0, ss1):
    c = lax.axis_index("c")
    s = lax.axis_index("s")
    lo = c * HALF0
    hi = HALF0 + c * (N_NODES - HALF0)
    sg = (sg0, sg1)
    ss = (ss0, ss1)

    def zrow(i, carry):
        for b in range(2):
            for k in range(4):
                rows2[b, i, pl.ds(k * 16, 16)] = jnp.zeros((16,), jnp.float32)
        return carry
    lax.fori_loop(0, CH, zrow, 0)
    for g in range(CH // 16):
        dloc[1, pl.ds(g * 16, 16)] = jnp.zeros((16,), jnp.int32)

    z0 = s * ZPT
    nfz, rz = divmod(ZPT, CH)
    for q in range(nfz):
        pltpu.sync_copy(rows2.at[0], acc.at[pl.ds(z0 + q * CH, CH)])
    if rz:
        pltpu.sync_copy(rows2.at[0, pl.ds(0, rz)],
                        acc.at[pl.ds(z0 + nfz * CH, rz)])
    plsc.subcore_barrier()

    def stage_and_gather(ci, b):
        row = s * NCH + ci
        pltpu.sync_copy(e2_hbm.at[row], ebuf.at[b])
        pltpu.sync_copy(w2_hbm.at[row], wbuf.at[b])
        pltpu.async_copy(x_hbm.at[ebuf.at[b, 0]], rows2.at[b], sg[b])

    stage_and_gather(0, 0)
    pltpu.async_copy(rows2.at[1], acc.at[dloc.at[1]], ss[1], add=True)

    def iter2(i2, carry):
        for b in range(2):
            ci = i2 * 2 + b
            pltpu.make_async_copy(x_hbm.at[ebuf.at[b, 0]], rows2.at[b],
                                  sg[b]).wait()
            pltpu.make_async_copy(rows2.at[1 - b], acc.at[dloc.at[1 - b]],
                                  ss[1 - b]).wait()
            cin = jnp.where(ci + 1 == NCH, 0, ci + 1)
            stage_and_gather(cin, 1 - b)
            for g in range(CH // 16):
                d16 = ebuf[b, 1, pl.ds(g * 16, 16)]
                inr = (d16 >= lo) & (d16 < hi)
                dloc[b, pl.ds(g * 16, 16)] = jnp.where(
                    inr, d16 - lo, HALF0 + (d16 & (DUMP - 1)))
            def blk(bi, carry2, _b=b):
                jb = bi * 16
                w16 = wbuf[_b, pl.ds(jb, 16)]
                for e in range(16):
                    j = jb + e
                    ws = _lane_splat(w16, e)
                    for k in range(4):
                        rows2[_b, j, pl.ds(k * 16, 16)] = (
                            rows2[_b, j, pl.ds(k * 16, 16)] * ws)
                return carry2
            lax.fori_loop(0, CH // 16, blk, 0)
            pltpu.async_copy(rows2.at[b], acc.at[dloc.at[b]], ss[b], add=True)
        return carry
    lax.fori_loop(0, NCH // 2, iter2, 0)

    pltpu.make_async_copy(x_hbm.at[ebuf.at[0, 0]], rows2.at[0], sg[0]).wait()
    pltpu.make_async_copy(rows2.at[1], acc.at[dloc.at[1]], ss[1]).wait()
    plsc.subcore_barrier()

    w0 = s * ROWS_PT
    g0 = c * HALF0 + w0
    nfw, rw = divmod(ROWS_PT, CH)
    for q in range(nfw):
        pltpu.sync_copy(acc.at[pl.ds(w0 + q * CH, CH)],
                        out_hbm.at[pl.ds(g0 + q * CH, CH)])
    if rw:
        pltpu.sync_copy(acc.at[pl.ds(w0 + nfw * CH, rw)],
                        out_hbm.at[pl.ds(g0 + nfw * CH, rw)])


@functools.cache
def _sc_layer_fn():
  return pl.kernel(
    _layer_body,
    out_type=jax.ShapeDtypeStruct((NPAD, EMB), jnp.float32),
    mesh=plsc.VectorSubcoreMesh(core_axis_name="c", subcore_axis_name="s",
                                num_cores=NC, num_subcores=NS),
    compiler_params=pltpu.CompilerParams(use_tc_tiling_on_sc=False),
    scratch_types=[
        pltpu.VMEM_SHARED((ACC_ROWS, EMB), jnp.float32),
        pltpu.VMEM((2, 2, CH), jnp.int32),
        pltpu.VMEM((2, CH), jnp.float32),
        pltpu.VMEM((2, CH), jnp.int32),
        pltpu.VMEM((2, CH, EMB), jnp.float32),
        pltpu.SemaphoreType.DMA,
        pltpu.SemaphoreType.DMA,
        pltpu.SemaphoreType.DMA,
        pltpu.SemaphoreType.DMA,
    ],
  )


def _mean_body(a, b, c, d, o):
    o[...] = (a[...] + b[...] + c[...] + d[...]) * 0.25


_mean = pl.pallas_call(
    _mean_body,
    grid=(250,),
    in_specs=[pl.BlockSpec((200, EMB), lambda i: (i, 0))] * 4,
    out_specs=pl.BlockSpec((200, EMB), lambda i: (i, 0)),
    out_shape=jax.ShapeDtypeStruct((N_NODES, EMB), jnp.float32),
)


def kernel(user_weight, item_weight, edge_index, edge_weight):
    x0 = jnp.concatenate(
        [user_weight, item_weight,
         jnp.zeros((NPAD - N_NODES, EMB), jnp.float32)], axis=0)
    src = edge_index[1]
    dst = edge_index[0]
    pad = EP - src.shape[0]
    src_p = jnp.concatenate([src, jnp.zeros((pad,), jnp.int32)])
    dst_p = jnp.concatenate([dst, jnp.full((pad,), N_NODES, jnp.int32)])
    w_p = jnp.concatenate([edge_weight, jnp.zeros((pad,), jnp.float32)])
    e2 = jnp.stack([src_p.reshape(EP // CH, CH),
                    dst_p.reshape(EP // CH, CH)], axis=1)
    w2 = w_p.reshape(EP // CH, CH)

    sc_layer = _sc_layer_fn()
    xs = [x0]
    cur = x0
    for _ in range(N_LAYERS):
        cur = sc_layer(cur, e2, w2)
        xs.append(cur)
    return _mean(*xs)


# speedup vs baseline: 3.8490x; 1.6116x over previous
import functools

import jax
import jax.numpy as jnp
from jax import lax
from jax.experimental import pallas as pl
from jax.experimental.pallas import tpu as pltpu
from jax.experimental.pallas import tpu_sc as plsc

N_USERS = 25000
N_ITEMS = 25000
N_NODES = 50000
EMB = 64
N_LAYERS = 3

NC = 2
NS = 16
CH = 128
EP = 802816
EPT = EP // NS
NCH = EPT // CH

HALF0 = 25088
ROWS_PT = HALF0 // NS
NPAD = 2 * HALF0
DUMP = 64
ACC_ROWS = HALF0 + DUMP
ZPT = ACC_ROWS // NS

_GATHER_DNUMS = lax.GatherDimensionNumbers(
    offset_dims=(), collapsed_slice_dims=(0,), start_index_map=(0,))


def _lane_splat(v16, e):
    idx = jnp.full((16, 1), e, jnp.int32)
    return lax.gather(v16, idx, _GATHER_DNUMS, (1,),
                      mode=lax.GatherScatterMode.PROMISE_IN_BOUNDS)


def _layer_body(x_hbm, e2_hbm, w2_hbm, out_hbm,
                acc, ebuf, wbuf, dloc, rows2, sg0, sg1, ss0, ss1):
    c = lax.axis_index("c")
    s = lax.axis_index("s")
    lo = c * HALF0
    hi = HALF0 + c * (N_NODES - HALF0)
    sg = (sg0, sg1)
    ss = (ss0, ss1)

    def zrow(i, carry):
        for b in range(2):
            for k in range(4):
                rows2[b, i, pl.ds(k * 16, 16)] = jnp.zeros((16,), jnp.float32)
        return carry
    lax.fori_loop(0, CH, zrow, 0)
    for g in range(CH // 16):
        dloc[1, pl.ds(g * 16, 16)] = jnp.zeros((16,), jnp.int32)

    z0 = s * ZPT
    nfz, rz = divmod(ZPT, CH)
    for q in range(nfz):
        pltpu.sync_copy(rows2.at[0], acc.at[pl.ds(z0 + q * CH, CH)])
    if rz:
        pltpu.sync_copy(rows2.at[0, pl.ds(0, rz)],
                        acc.at[pl.ds(z0 + nfz * CH, rz)])
    plsc.subcore_barrier()

    def stage_and_gather(ci, b):
        row = s * NCH + ci
        pltpu.sync_copy(e2_hbm.at[row], ebuf.at[b])
        pltpu.sync_copy(w2_hbm.at[row], wbuf.at[b])
        pltpu.async_copy(x_hbm.at[ebuf.at[b, 0]], rows2.at[b], sg[b])

    stage_and_gather(0, 0)
    pltpu.async_copy(rows2.at[1], acc.at[dloc.at[1]], ss[1], add=True)

    def iter2(i2, carry):
        for b in range(2):
            ci = i2 * 2 + b
            pltpu.make_async_copy(x_hbm.at[ebuf.at[b, 0]], rows2.at[b],
                                  sg[b]).wait()
            pltpu.make_async_copy(rows2.at[1 - b], acc.at[dloc.at[1 - b]],
                                  ss[1 - b]).wait()
            cin = jnp.where(ci + 1 == NCH, 0, ci + 1)
            stage_and_gather(cin, 1 - b)
            for g in range(CH // 16):
                d16 = ebuf[b, 1, pl.ds(g * 16, 16)]
                inr = (d16 >= lo) & (d16 < hi)
                dloc[b, pl.ds(g * 16, 16)] = jnp.where(
                    inr, d16 - lo, HALF0 + (d16 & (DUMP - 1)))
            def blk(bi, carry2, _b=b):
                jb = bi * 16
                w16 = wbuf[_b, pl.ds(jb, 16)]
                for e in range(0, 16, 2):
                    j0 = jb + e
                    j1 = jb + e + 1
                    ws0 = _lane_splat(w16, e)
                    ws1 = _lane_splat(w16, e + 1)
                    vals = [rows2[_b, j0, pl.ds(k * 16, 16)] for k in range(4)]
                    vals += [rows2[_b, j1, pl.ds(k * 16, 16)] for k in range(4)]
                    prods = [v * ws0 for v in vals[:4]] + \
                            [v * ws1 for v in vals[4:]]
                    for k in range(4):
                        rows2[_b, j0, pl.ds(k * 16, 16)] = prods[k]
                    for k in range(4):
                        rows2[_b, j1, pl.ds(k * 16, 16)] = prods[4 + k]
                return carry2
            lax.fori_loop(0, CH // 16, blk, 0)
            pltpu.async_copy(rows2.at[b], acc.at[dloc.at[b]], ss[b], add=True)
        return carry
    lax.fori_loop(0, NCH // 2, iter2, 0)

    pltpu.make_async_copy(x_hbm.at[ebuf.at[0, 0]], rows2.at[0], sg[0]).wait()
    pltpu.make_async_copy(rows2.at[1], acc.at[dloc.at[1]], ss[1]).wait()
    plsc.subcore_barrier()

    w0 = s * ROWS_PT
    g0 = c * HALF0 + w0
    nfw, rw = divmod(ROWS_PT, CH)
    for q in range(nfw):
        pltpu.sync_copy(acc.at[pl.ds(w0 + q * CH, CH)],
                        out_hbm.at[pl.ds(g0 + q * CH, CH)])
    if rw:
        pltpu.sync_copy(acc.at[pl.ds(w0 + nfw * CH, rw)],
                        out_hbm.at[pl.ds(g0 + nfw * CH, rw)])


@functools.cache
def _sc_layer_fn():
  return pl.kernel(
    _layer_body,
    out_type=jax.ShapeDtypeStruct((NPAD, EMB), jnp.float32),
    mesh=plsc.VectorSubcoreMesh(core_axis_name="c", subcore_axis_name="s",
                                num_cores=NC, num_subcores=NS),
    compiler_params=pltpu.CompilerParams(use_tc_tiling_on_sc=False),
    scratch_types=[
        pltpu.VMEM_SHARED((ACC_ROWS, EMB), jnp.float32),
        pltpu.VMEM((2, 2, CH), jnp.int32),
        pltpu.VMEM((2, CH), jnp.float32),
        pltpu.VMEM((2, CH), jnp.int32),
        pltpu.VMEM((2, CH, EMB), jnp.float32),
        pltpu.SemaphoreType.DMA,
        pltpu.SemaphoreType.DMA,
        pltpu.SemaphoreType.DMA,
        pltpu.SemaphoreType.DMA,
    ],
  )


def _mean_body(a, b, c, d, o):
    o[...] = (a[...] + b[...] + c[...] + d[...]) * 0.25


_mean = pl.pallas_call(
    _mean_body,
    grid=(250,),
    in_specs=[pl.BlockSpec((200, EMB), lambda i: (i, 0))] * 4,
    out_specs=pl.BlockSpec((200, EMB), lambda i: (i, 0)),
    out_shape=jax.ShapeDtypeStruct((N_NODES, EMB), jnp.float32),
)


def kernel(user_weight, item_weight, edge_index, edge_weight):
    x0 = jnp.concatenate(
        [user_weight, item_weight,
         jnp.zeros((NPAD - N_NODES, EMB), jnp.float32)], axis=0)
    src = edge_index[1]
    dst = edge_index[0]
    pad = EP - src.shape[0]
    src_p = jnp.concatenate([src, jnp.zeros((pad,), jnp.int32)])
    dst_p = jnp.concatenate([dst, jnp.full((pad,), N_NODES, jnp.int32)])
    w_p = jnp.concatenate([edge_weight, jnp.zeros((pad,), jnp.float32)])
    e2 = jnp.stack([src_p.reshape(EP // CH, CH),
                    dst_p.reshape(EP // CH, CH)], axis=1)
    w2 = w_p.reshape(EP // CH, CH)

    sc_layer = _sc_layer_fn()
    xs = [x0]
    cur = x0
    for _ in range(N_LAYERS):
        cur = sc_layer(cur, e2, w2)
        xs.append(cur)
    return _mean(*xs)


# speedup vs baseline: 5.2729x; 1.3699x over previous
import functools

import jax
import jax.numpy as jnp
from jax import lax
from jax.experimental import pallas as pl
from jax.experimental.pallas import tpu as pltpu
from jax.experimental.pallas import tpu_sc as plsc

N_USERS = 25000
N_ITEMS = 25000
N_NODES = 50000
EMB = 64
N_LAYERS = 3

NC = 2
NS = 16
NW = NC * NS
CH = 128
EP = 802816
EPT32 = EP // NW
PNCH = EPT32 // CH
CAPB = PNCH + 4

HALF0 = 25088
ROWS_PT = HALF0 // NS
NPAD = 2 * HALF0
DUMP = 16
ACC_ROWS = HALF0 + DUMP
ZPT = ACC_ROWS // NS

_GATHER_DNUMS = lax.GatherDimensionNumbers(
    offset_dims=(), collapsed_slice_dims=(0,), start_index_map=(0,))


def _lane_splat(v16, e):
    idx = jnp.full((16, 1), e, jnp.int32)
    return lax.gather(v16, idx, _GATHER_DNUMS, (1,),
                      mode=lax.GatherScatterMode.PROMISE_IN_BOUNDS)


def _part_body(e2_hbm, w2_hbm, eo_hbm, wo_hbm, cnt_hbm,
               ebuf, wbuf, cbuf, srA, dsA, wrA, srB, dsB, wrB):
    c = lax.axis_index("c")
    s = lax.axis_index("s")
    wid = c * NS + s
    iota = lax.iota(jnp.int32, 16)
    zi = jnp.zeros((16,), jnp.int32)
    zf = jnp.zeros((16,), jnp.float32)

    for g in range(16):
        srA[pl.ds(g * 16, 16)] = zi
        dsA[pl.ds(g * 16, 16)] = zi
        srB[pl.ds(g * 16, 16)] = zi
        dsB[pl.ds(g * 16, 16)] = zi
        wrA[pl.ds(g * 16, 16)] = zf
        wrB[pl.ds(g * 16, 16)] = zf

    rings = ((srA, dsA, wrA), (srB, dsB, wrB))

    def flush(h, cur):
        sr, ds_, wr = rings[h]
        rb = (cur >> 7) & 1
        blk = cur >> 7
        pltpu.sync_copy(sr.at[pl.ds(rb * 128, 128)],
                        eo_hbm.at[h, wid, blk, 0])
        pltpu.sync_copy(ds_.at[pl.ds(rb * 128, 128)],
                        eo_hbm.at[h, wid, blk, 1])
        pltpu.sync_copy(wr.at[pl.ds(rb * 128, 128)],
                        wo_hbm.at[h, wid, blk])

    def chunk(i, curs):
        curA, curB = curs
        row = wid * PNCH + i
        pltpu.sync_copy(e2_hbm.at[row], ebuf)
        pltpu.sync_copy(w2_hbm.at[row], wbuf)
        for g in range(CH // 16):
            o = g * 16
            src16 = ebuf[0, pl.ds(o, 16)]
            dst16 = ebuf[1, pl.ds(o, 16)]
            w16 = wbuf[pl.ds(o, 16)]
            mA = dst16 < HALF0
            dloc16 = jnp.where(mA, dst16, dst16 - HALF0)
            for h, m, cur in ((0, mA, curA), (1, ~mA, curB)):
                sr, ds_, wr = rings[h]
                cm = plsc.cumsum(jnp.where(m, 1, 0))
                pos = (cur + cm - 1) & 255
                plsc.store_scatter(sr, [pos], src16, mask=m)
                plsc.store_scatter(ds_, [pos], dloc16, mask=m)
                plsc.store_scatter(wr, [pos], w16, mask=m)
                ncur = cur + jnp.max(cm)

                @pl.when((ncur >> 7) != (cur >> 7))
                def _(h=h, cur=cur):
                    flush(h, cur)
                if h == 0:
                    curA = ncur
                else:
                    curB = ncur
        return (curA, curB)

    curA, curB = lax.fori_loop(0, PNCH, chunk, (jnp.int32(0), jnp.int32(0)))

    nblks = []
    for h, cur in ((0, curA), (1, curB)):
        sr, ds_, wr = rings[h]
        rem = cur & 127
        for g in range(CH // 16):
            pos = (cur + g * 16 + iota) & 255
            mm = (g * 16 + iota) < (128 - rem)
            plsc.store_scatter(wr, [pos], zf, mask=mm)
        flush(h, cur)
        nblks.append((cur >> 7) + 1)

    cv = (jnp.where(iota == 0, nblks[0], 0)
          + jnp.where(iota == 1, nblks[1], 0))
    cbuf[pl.ds(0, 16)] = cv
    pltpu.sync_copy(cbuf, cnt_hbm.at[wid])


@functools.cache
def _partition_fn():
  return pl.kernel(
    _part_body,
    out_type=(
        jax.ShapeDtypeStruct((2, NW, CAPB, 2, CH), jnp.int32),
        jax.ShapeDtypeStruct((2, NW, CAPB, CH), jnp.float32),
        jax.ShapeDtypeStruct((NW, 16), jnp.int32),
    ),
    mesh=plsc.VectorSubcoreMesh(core_axis_name="c", subcore_axis_name="s",
                                num_cores=NC, num_subcores=NS),
    compiler_params=pltpu.CompilerParams(use_tc_tiling_on_sc=False,
                                         needs_layout_passes=False),
    scratch_types=[
        pltpu.VMEM((2, CH), jnp.int32),
        pltpu.VMEM((CH,), jnp.float32),
        pltpu.VMEM((16,), jnp.int32),
        pltpu.VMEM((256,), jnp.int32),
        pltpu.VMEM((256,), jnp.int32),
        pltpu.VMEM((256,), jnp.float32),
        pltpu.VMEM((256,), jnp.int32),
        pltpu.VMEM((256,), jnp.int32),
        pltpu.VMEM((256,), jnp.float32),
    ],
  )


def _layer_body(x_hbm, eo_hbm, wo_hbm, cnt_hbm, out_hbm,
                acc, ebuf, wbuf, dloc, rows2, cntv, sg0, sg1, ss0, ss1):
    c = lax.axis_index("c")
    s = lax.axis_index("s")
    sg = (sg0, sg1)
    ss = (ss0, ss1)
    iota = lax.iota(jnp.int32, 16)
    t1 = s
    t2 = NS + s

    pltpu.sync_copy(cnt_hbm.at[t1], cntv.at[0])
    pltpu.sync_copy(cnt_hbm.at[t2], cntv.at[1])
    n1 = jnp.max(jnp.where(iota == c, cntv[0, pl.ds(0, 16)], 0))
    n2 = jnp.max(jnp.where(iota == c, cntv[1, pl.ds(0, 16)], 0))
    nbt = n1 + n2
    nbt_pad = nbt + (nbt & 1)

    def zrow(i, carry):
        for b in range(2):
            for k in range(4):
                rows2[b, i, pl.ds(k * 16, 16)] = jnp.zeros((16,), jnp.float32)
        return carry
    lax.fori_loop(0, CH, zrow, 0)
    for g in range(CH // 16):
        dloc[1, pl.ds(g * 16, 16)] = jnp.zeros((16,), jnp.int32)

    z0 = s * ZPT
    nfz, rz = divmod(ZPT, CH)
    for q in range(nfz):
        pltpu.sync_copy(rows2.at[0], acc.at[pl.ds(z0 + q * CH, CH)])
    if rz:
        pltpu.sync_copy(rows2.at[0, pl.ds(0, rz)],
                        acc.at[pl.ds(z0 + nfz * CH, rz)])
    plsc.subcore_barrier()

    def stage_and_gather(i, b):
        iv = jnp.minimum(i, nbt - 1)
        t = jnp.where(iv < n1, t1, t2)
        blk = jnp.where(iv < n1, iv, iv - n1)
        pltpu.sync_copy(eo_hbm.at[c, t, blk], ebuf.at[b])
        pltpu.sync_copy(wo_hbm.at[c, t, blk], wbuf.at[b])
        pltpu.async_copy(x_hbm.at[ebuf.at[b, 0]], rows2.at[b], sg[b])

    stage_and_gather(jnp.int32(0), 0)
    pltpu.async_copy(rows2.at[1], acc.at[dloc.at[1]], ss[1], add=True)

    dump_vec = HALF0 + (iota & (DUMP - 1))

    def iter2(i2, carry):
        for b in range(2):
            ci = i2 * 2 + b
            pltpu.make_async_copy(x_hbm.at[ebuf.at[b, 0]], rows2.at[b],
                                  sg[b]).wait()
            pltpu.make_async_copy(rows2.at[1 - b], acc.at[dloc.at[1 - b]],
                                  ss[1 - b]).wait()
            stage_and_gather(ci + 1, 1 - b)
            valid = ci < nbt
            for g in range(CH // 16):
                d16 = ebuf[b, 1, pl.ds(g * 16, 16)]
                dloc[b, pl.ds(g * 16, 16)] = jnp.where(valid, d16, dump_vec)
            def blk_fn(bi, carry2, _b=b):
                jb = bi * 16
                w16 = wbuf[_b, pl.ds(jb, 16)]
                for e in range(0, 16, 2):
                    j0 = jb + e
                    j1 = jb + e + 1
                    ws0 = _lane_splat(w16, e)
                    ws1 = _lane_splat(w16, e + 1)
                    vals = [rows2[_b, j0, pl.ds(k * 16, 16)]
                            for k in range(4)]
                    vals += [rows2[_b, j1, pl.ds(k * 16, 16)]
                             for k in range(4)]
                    prods = [v * ws0 for v in vals[:4]] + \
                            [v * ws1 for v in vals[4:]]
                    for k in range(4):
                        rows2[_b, j0, pl.ds(k * 16, 16)] = prods[k]
                    for k in range(4):
                        rows2[_b, j1, pl.ds(k * 16, 16)] = prods[4 + k]
                return carry2
            lax.fori_loop(0, CH // 16, blk_fn, 0)
            pltpu.async_copy(rows2.at[b], acc.at[dloc.at[b]], ss[b], add=True)
        return carry
    lax.fori_loop(0, nbt_pad // 2, iter2, 0)

    pltpu.make_async_copy(x_hbm.at[ebuf.at[0, 0]], rows2.at[0], sg[0]).wait()
    pltpu.make_async_copy(rows2.at[1], acc.at[dloc.at[1]], ss[1]).wait()
    plsc.subcore_barrier()

    w0 = s * ROWS_PT
    g0 = c * HALF0 + w0
    nfw, rw = divmod(ROWS_PT, CH)
    for q in range(nfw):
        pltpu.sync_copy(acc.at[pl.ds(w0 + q * CH, CH)],
                        out_hbm.at[pl.ds(g0 + q * CH, CH)])
    if rw:
        pltpu.sync_copy(acc.at[pl.ds(w0 + nfw * CH, rw)],
                        out_hbm.at[pl.ds(g0 + nfw * CH, rw)])


@functools.cache
def _sc_layer_fn():
  return pl.kernel(
    _layer_body,
    out_type=jax.ShapeDtypeStruct((NPAD, EMB), jnp.float32),
    mesh=plsc.VectorSubcoreMesh(core_axis_name="c", subcore_axis_name="s",
                                num_cores=NC, num_subcores=NS),
    compiler_params=pltpu.CompilerParams(use_tc_tiling_on_sc=False,
                                         needs_layout_passes=False),
    scratch_types=[
        pltpu.VMEM_SHARED((ACC_ROWS, EMB), jnp.float32),
        pltpu.VMEM((2, 2, CH), jnp.int32),
        pltpu.VMEM((2, CH), jnp.float32),
        pltpu.VMEM((2, CH), jnp.int32),
        pltpu.VMEM((2, CH, EMB), jnp.float32),
        pltpu.VMEM((2, 16), jnp.int32),
        pltpu.SemaphoreType.DMA,
        pltpu.SemaphoreType.DMA,
        pltpu.SemaphoreType.DMA,
        pltpu.SemaphoreType.DMA,
    ],
  )


def _mean_body(a, b, c, d, o):
    o[...] = (a[...] + b[...] + c[...] + d[...]) * 0.25


_mean = pl.pallas_call(
    _mean_body,
    grid=(250,),
    in_specs=[pl.BlockSpec((200, EMB), lambda i: (i, 0))] * 4,
    out_specs=pl.BlockSpec((200, EMB), lambda i: (i, 0)),
    out_shape=jax.ShapeDtypeStruct((N_NODES, EMB), jnp.float32),
)


def kernel(user_weight, item_weight, edge_index, edge_weight):
    x0 = jnp.concatenate(
        [user_weight, item_weight,
         jnp.zeros((NPAD - N_NODES, EMB), jnp.float32)], axis=0)
    src = edge_index[1]
    dst = edge_index[0]
    pad = EP - src.shape[0]
    src_p = jnp.concatenate([src, jnp.zeros((pad,), jnp.int32)])
    dst_p = jnp.concatenate([dst, jnp.full((pad,), N_NODES, jnp.int32)])
    w_p = jnp.concatenate([edge_weight, jnp.zeros((pad,), jnp.float32)])
    e2 = jnp.stack([src_p.reshape(EP // CH, CH),
                    dst_p.reshape(EP // CH, CH)], axis=1)
    w2 = w_p.reshape(EP // CH, CH)

    eo, wo, cnt = _partition_fn()(e2, w2)

    sc_layer = _sc_layer_fn()
    xs = [x0]
    cur = x0
    for _ in range(N_LAYERS):
        cur = sc_layer(cur, eo, wo, cnt)
        xs.append(cur)
    return _mean(*xs)
